# Initial kernel scaffold; baseline (speedup 1.0000x reference)
import jax
import jax.numpy as jnp
from jax.experimental import pallas as pl


def kernel(x, edge_index, W1, b1, W2, b2, Wlin, blin):
    raise NotImplementedError("write your pallas kernel here")


# speedup vs baseline: 19.7454x; 19.7454x over previous
import functools

import jax
import jax.numpy as jnp
from jax import lax
from jax.experimental import pallas as pl
from jax.experimental.pallas import tpu as pltpu
from jax.experimental.pallas import tpu_sc as plsc

_NC = 2
_NS = 16
_NW = _NC * _NS
_L = 16


def _sc_mesh():
    return plsc.VectorSubcoreMesh(core_axis_name="c", subcore_axis_name="s")


def _fill(ref, n, value16):
    def body(i, _):
        ref[pl.ds(i * _L, _L)] = value16
        return 0
    lax.fori_loop(0, n // _L, body, 0)


def _sc_degree(dst_r, n_pad, k_chunk, n_chunks):
    rows_per_tile = n_pad // _NS

    @functools.partial(
        pl.kernel,
        out_type=jax.ShapeDtypeStruct((_NC, n_pad), jnp.float32),
        mesh=_sc_mesh(),
        scratch_types=[
            pltpu.VMEM((n_chunks, k_chunk), jnp.int32),
            pltpu.VMEM((k_chunk,), jnp.float32),
            pltpu.VMEM((rows_per_tile,), jnp.float32),
            pltpu.VMEM_SHARED((n_pad,), jnp.float32),
        ],
    )
    def k(dst_hbm, out_hbm, dst_v, ones_v, zb, acc_sp):
        c = lax.axis_index("c")
        s = lax.axis_index("s")
        wid = c * _NS + s
        base = s * rows_per_tile
        _fill(ones_v, k_chunk, jnp.ones((_L,), jnp.float32))
        _fill(zb, rows_per_tile, jnp.zeros((_L,), jnp.float32))
        pltpu.sync_copy(zb, acc_sp.at[pl.ds(base, rows_per_tile)])
        plsc.subcore_barrier()
        pltpu.sync_copy(dst_hbm.at[wid], dst_v)

        def body(j, _):
            pltpu.sync_copy(ones_v, acc_sp.at[dst_v.at[j]], add=True)
            return 0
        lax.fori_loop(0, n_chunks, body, 0)

        plsc.subcore_barrier()
        pltpu.sync_copy(acc_sp.at[pl.ds(base, rows_per_tile)],
                        out_hbm.at[c, pl.ds(base, rows_per_tile)])

    return k(dst_r)


def _sc_segment_sum(xwp, src_r, dst_r, n_pad, k_chunk, n_chunks):
    d = xwp.shape[1]
    rows_per_tile = n_pad // _NS
    zero_copies = rows_per_tile // k_chunk

    @functools.partial(
        pl.kernel,
        out_type=jax.ShapeDtypeStruct((_NC, n_pad, d), jnp.float32),
        mesh=_sc_mesh(),
        scratch_types=[
            pltpu.VMEM((n_chunks, k_chunk), jnp.int32),
            pltpu.VMEM((n_chunks, k_chunk), jnp.int32),
            pltpu.VMEM((k_chunk, d), jnp.float32),
            pltpu.VMEM((k_chunk, d), jnp.float32),
            pltpu.VMEM_SHARED((n_pad, d), jnp.float32),
            pltpu.SemaphoreType.DMA,
            pltpu.SemaphoreType.DMA,
        ],
    )
    def k(xw_hbm, src_hbm, dst_hbm, out_hbm,
          src_v, dst_v, rows0, rows1, acc_sp, sem0, sem1):
        c = lax.axis_index("c")
        s = lax.axis_index("s")
        wid = c * _NS + s
        base = s * rows_per_tile

        zero16 = jnp.zeros((_L,), jnp.float32)

        def zrow(i, _):
            def zcol(j, _):
                rows0[i, pl.ds(j * _L, _L)] = zero16
                return 0
            lax.fori_loop(0, d // _L, zcol, 0)
            return 0
        lax.fori_loop(0, k_chunk, zrow, 0)
        for t in range(zero_copies):
            pltpu.sync_copy(rows0, acc_sp.at[pl.ds(base + t * k_chunk, k_chunk)])
        plsc.subcore_barrier()

        pltpu.sync_copy(src_hbm.at[wid], src_v)
        pltpu.sync_copy(dst_hbm.at[wid], dst_v)

        def body(j, _):
            pltpu.async_copy(xw_hbm.at[src_v.at[j]], rows0, sem0).wait()
            pltpu.sync_copy(rows0, acc_sp.at[dst_v.at[j]], add=True)
            return 0
        lax.fori_loop(0, n_chunks, body, 0)

        plsc.subcore_barrier()
        pltpu.sync_copy(acc_sp.at[pl.ds(base, rows_per_tile)],
                        out_hbm.at[c, pl.ds(base, rows_per_tile)])

    return k(xwp, src_r, dst_r)


def _tc_prescale(x, w1, deg_t, rows):
    n, d_in = x.shape
    d_out = w1.shape[1]

    def body(x_b, w_b, deg_b, xwp_b, dinv_b):
        deg = deg_b[:, 0:1] + deg_b[:, 1:2] + 1.0
        dinv = lax.rsqrt(deg)
        xw = jnp.dot(x_b[...], w_b[...], preferred_element_type=jnp.float32)
        xwp_b[...] = xw * dinv
        dinv_b[...] = dinv

    return pl.pallas_call(
        body,
        grid=(n // rows,),
        in_specs=[
            pl.BlockSpec((rows, d_in), lambda i: (i, 0)),
            pl.BlockSpec((d_in, d_out), lambda i: (0, 0)),
            pl.BlockSpec((rows, _NC), lambda i: (i, 0)),
        ],
        out_specs=[
            pl.BlockSpec((rows, d_out), lambda i: (i, 0)),
            pl.BlockSpec((rows, 1), lambda i: (i, 0)),
        ],
        out_shape=[
            jax.ShapeDtypeStruct((n, d_out), jnp.float32),
            jax.ShapeDtypeStruct((n, 1), jnp.float32),
        ],
    )(x, w1, deg_t)


def _tc_mid(acc, xwp, dinv, b_in, w, rows):
    n, d = xwp.shape
    n_pad = acc.shape[1]
    d_out = w.shape[1]

    def body(a0_b, a1_b, xwp_b, dinv_b, b_b, w_b, out_b):
        h = jnp.maximum(
            (a0_b[0] + a1_b[0] + xwp_b[...]) * dinv_b[...] + b_b[...], 0.0)
        out_b[...] = jnp.dot(h, w_b[...],
                             preferred_element_type=jnp.float32) * dinv_b[...]

    return pl.pallas_call(
        body,
        grid=(n // rows,),
        in_specs=[
            pl.BlockSpec((1, rows, d), lambda i: (0, i, 0)),
            pl.BlockSpec((1, rows, d), lambda i: (1, i, 0)),
            pl.BlockSpec((rows, d), lambda i: (i, 0)),
            pl.BlockSpec((rows, 1), lambda i: (i, 0)),
            pl.BlockSpec((1, d), lambda i: (0, 0)),
            pl.BlockSpec((d, d_out), lambda i: (0, 0)),
        ],
        out_specs=pl.BlockSpec((rows, d_out), lambda i: (i, 0)),
        out_shape=jax.ShapeDtypeStruct((n, d_out), jnp.float32),
    )(acc, acc, xwp, dinv, b_in.reshape(1, d), w)


def _tc_final(acc, xwp, dinv, b_in, w, b_out, rows):
    n, d = xwp.shape
    d_out = w.shape[1]

    def body(a0_b, a1_b, xwp_b, dinv_b, b_b, w_b, bo_b, out_b):
        h = jnp.maximum(
            (a0_b[0] + a1_b[0] + xwp_b[...]) * dinv_b[...] + b_b[...], 0.0)
        z = jnp.dot(h, w_b[...], preferred_element_type=jnp.float32) + bo_b[...]
        out_b[...] = jax.nn.sigmoid(z)

    return pl.pallas_call(
        body,
        grid=(n // rows,),
        in_specs=[
            pl.BlockSpec((1, rows, d), lambda i: (0, i, 0)),
            pl.BlockSpec((1, rows, d), lambda i: (1, i, 0)),
            pl.BlockSpec((rows, d), lambda i: (i, 0)),
            pl.BlockSpec((rows, 1), lambda i: (i, 0)),
            pl.BlockSpec((1, d), lambda i: (0, 0)),
            pl.BlockSpec((d, d_out), lambda i: (0, 0)),
            pl.BlockSpec((1, d_out), lambda i: (0, 0)),
        ],
        out_specs=pl.BlockSpec((rows, d_out), lambda i: (i, 0)),
        out_shape=jax.ShapeDtypeStruct((n, d_out), jnp.float32),
    )(acc, acc, xwp, dinv, b_in.reshape(1, d), w, b_out.reshape(1, d_out))


def kernel(x, edge_index, W1, b1, W2, b2, Wlin, blin):
    n, _ = x.shape
    e = edge_index.shape[1]

    per_tile = e // _NW
    assert e % _NW == 0
    k_chunk = 80
    assert per_tile % k_chunk == 0
    n_chunks = per_tile // k_chunk
    n_pad = -(-n // (_NS * k_chunk)) * (_NS * k_chunk)

    src_r = edge_index[0].reshape(_NW, n_chunks, k_chunk)
    dst_r = edge_index[1].reshape(_NW, n_chunks, k_chunk)

    rows = 1000
    assert n % rows == 0

    deg_parts = _sc_degree(dst_r, n_pad, k_chunk, n_chunks)
    deg_t = deg_parts.T[:n]

    xwp1, dinv = _tc_prescale(x, W1, deg_t, rows)
    acc1 = _sc_segment_sum(xwp1, src_r, dst_r, n_pad, k_chunk, n_chunks)
    xwp2 = _tc_mid(acc1, xwp1, dinv, b1, W2, rows)
    acc2 = _sc_segment_sum(xwp2, src_r, dst_r, n_pad, k_chunk, n_chunks)
    return _tc_final(acc2, xwp2, dinv, b2, Wlin, blin, rows)


# speedup vs baseline: 27.9659x; 1.4163x over previous
import functools

import jax
import jax.numpy as jnp
from jax import lax
from jax.experimental import pallas as pl
from jax.experimental.pallas import tpu as pltpu
from jax.experimental.pallas import tpu_sc as plsc

_NC = 2
_NS = 16
_NW = _NC * _NS
_L = 16


def _sc_mesh():
    return plsc.VectorSubcoreMesh(core_axis_name="c", subcore_axis_name="s")


def _fill(ref, n, value16):
    def body(i, _):
        ref[pl.ds(i * _L, _L)] = value16
        return 0
    lax.fori_loop(0, n // _L, body, 0)


def _sc_degree(dst_r, n_pad, k_chunk, n_chunks):
    rows_per_tile = n_pad // _NS
    zb_n = -(-rows_per_tile // _L) * _L

    @functools.partial(
        pl.kernel,
        out_type=jax.ShapeDtypeStruct((_NC, n_pad), jnp.float32),
        mesh=_sc_mesh(),
        scratch_types=[
            pltpu.VMEM((n_chunks, k_chunk), jnp.int32),
            pltpu.VMEM((k_chunk,), jnp.float32),
            pltpu.VMEM((zb_n,), jnp.float32),
            pltpu.VMEM_SHARED((n_pad,), jnp.float32),
        ],
    )
    def k(dst_hbm, out_hbm, dst_v, ones_v, zb, acc_sp):
        c = lax.axis_index("c")
        s = lax.axis_index("s")
        wid = c * _NS + s
        base = s * rows_per_tile
        _fill(ones_v, k_chunk, jnp.ones((_L,), jnp.float32))
        _fill(zb, zb_n, jnp.zeros((_L,), jnp.float32))
        pltpu.sync_copy(zb.at[pl.ds(0, rows_per_tile)],
                        acc_sp.at[pl.ds(base, rows_per_tile)])
        plsc.subcore_barrier()
        pltpu.sync_copy(dst_hbm.at[wid], dst_v)

        def body(j, _):
            pltpu.sync_copy(ones_v, acc_sp.at[dst_v.at[j]], add=True)
            return 0
        lax.fori_loop(0, n_chunks, body, 0)

        plsc.subcore_barrier()
        pltpu.sync_copy(acc_sp.at[pl.ds(base, rows_per_tile)],
                        out_hbm.at[c, pl.ds(base, rows_per_tile)])

    return k(dst_r)


def _sc_segment_sum(xwp, src_r, dst_r, n_pad, k_chunk, n_chunks, n_blocks):
    d = xwp.shape[1]
    rows_per_tile = n_pad // _NS
    sb = n_chunks // n_blocks

    @functools.partial(
        pl.kernel,
        out_type=jax.ShapeDtypeStruct((_NC, n_pad, d), jnp.float32),
        mesh=_sc_mesh(),
        scratch_types=[
            pltpu.VMEM((sb, k_chunk), jnp.int32),
            pltpu.VMEM((sb, k_chunk), jnp.int32),
            pltpu.VMEM((n_chunks, k_chunk), jnp.int32),
            pltpu.VMEM((k_chunk, d), jnp.float32),
            pltpu.VMEM((k_chunk, d), jnp.float32),
            pltpu.VMEM_SHARED((n_pad, d), jnp.float32),
            pltpu.SemaphoreType.DMA,
            pltpu.SemaphoreType.DMA,
            pltpu.SemaphoreType.DMA,
            pltpu.SemaphoreType.DMA,
        ],
    )
    def k(xw_hbm, src_hbm, dst_hbm, out_hbm,
          srcb0, srcb1, dst_v, rows0, rows1, acc_sp, ssem0, ssem1, sem0, sem1):
        c = lax.axis_index("c")
        s = lax.axis_index("s")
        wid = c * _NS + s
        base = s * rows_per_tile

        zero16 = jnp.zeros((_L,), jnp.float32)

        def zrow(i, _):
            def zcol(j, _):
                rows0[i, pl.ds(j * _L, _L)] = zero16
                return 0
            lax.fori_loop(0, d // _L, zcol, 0)
            return 0
        lax.fori_loop(0, k_chunk, zrow, 0)
        zc = (k_chunk // 8) * 8
        done = 0
        while done < rows_per_tile:
            nrow = min(zc, rows_per_tile - done)
            pltpu.sync_copy(rows0.at[pl.ds(0, nrow)],
                            acc_sp.at[pl.ds(base + done, nrow)])
            done += nrow
        plsc.subcore_barrier()

        pltpu.sync_copy(dst_hbm.at[wid], dst_v)

        srcbs = [(srcb0, ssem0), (srcb1, ssem1)]
        pltpu.async_copy(src_hbm.at[wid, pl.ds(0, sb)], srcb0, ssem0)
        for b in range(n_blocks):
            srcb, ssem = srcbs[b % 2]
            pltpu.make_async_copy(src_hbm.at[wid, pl.ds(b * sb, sb)],
                                  srcb, ssem).wait()
            if b + 1 < n_blocks:
                nsrcb, nssem = srcbs[(b + 1) % 2]
                pltpu.async_copy(src_hbm.at[wid, pl.ds((b + 1) * sb, sb)],
                                 nsrcb, nssem)

            pltpu.async_copy(xw_hbm.at[srcb.at[0]], rows0, sem0)

            def outer(i, _):
                j0 = i * 2

                def dchunk(jj):
                    return dst_v.at[b * sb + jj]

                pltpu.make_async_copy(
                    xw_hbm.at[srcb.at[j0]], rows0, sem0).wait()

                @pl.when(j0 + 1 < sb)
                def _():
                    pltpu.async_copy(xw_hbm.at[srcb.at[j0 + 1]], rows1, sem1)
                pltpu.sync_copy(rows0, acc_sp.at[dchunk(j0)], add=True)

                @pl.when(j0 + 1 < sb)
                def _():
                    pltpu.make_async_copy(
                        xw_hbm.at[srcb.at[j0 + 1]], rows1, sem1).wait()

                    @pl.when(j0 + 2 < sb)
                    def _():
                        pltpu.async_copy(
                            xw_hbm.at[srcb.at[j0 + 2]], rows0, sem0)
                    pltpu.sync_copy(rows1, acc_sp.at[dchunk(j0 + 1)], add=True)
                return 0
            lax.fori_loop(0, (sb + 1) // 2, outer, 0)

        plsc.subcore_barrier()
        pltpu.sync_copy(acc_sp.at[pl.ds(base, rows_per_tile)],
                        out_hbm.at[c, pl.ds(base, rows_per_tile)])

    return k(xwp, src_r, dst_r)


def _tc_prescale(x, w1, deg_t, rows):
    n, d_in = x.shape
    d_out = w1.shape[1]

    def body(x_b, w_b, deg_b, xwp_b, dinv_b):
        deg = deg_b[:, 0:1] + deg_b[:, 1:2] + 1.0
        dinv = lax.rsqrt(deg)
        xw = jnp.dot(x_b[...], w_b[...], preferred_element_type=jnp.float32)
        xwp_b[...] = xw * dinv
        dinv_b[...] = dinv

    return pl.pallas_call(
        body,
        grid=(n // rows,),
        in_specs=[
            pl.BlockSpec((rows, d_in), lambda i: (i, 0)),
            pl.BlockSpec((d_in, d_out), lambda i: (0, 0)),
            pl.BlockSpec((rows, _NC), lambda i: (i, 0)),
        ],
        out_specs=[
            pl.BlockSpec((rows, d_out), lambda i: (i, 0)),
            pl.BlockSpec((rows, 1), lambda i: (i, 0)),
        ],
        out_shape=[
            jax.ShapeDtypeStruct((n, d_out), jnp.float32),
            jax.ShapeDtypeStruct((n, 1), jnp.float32),
        ],
    )(x, w1, deg_t)


def _tc_mid(acc, xwp, dinv, b_in, w, rows):
    n, d = xwp.shape
    n_pad = acc.shape[1]
    d_out = w.shape[1]

    def body(a0_b, a1_b, xwp_b, dinv_b, b_b, w_b, out_b):
        h = jnp.maximum(
            (a0_b[0] + a1_b[0] + xwp_b[...]) * dinv_b[...] + b_b[...], 0.0)
        out_b[...] = jnp.dot(h, w_b[...],
                             preferred_element_type=jnp.float32) * dinv_b[...]

    return pl.pallas_call(
        body,
        grid=(n // rows,),
        in_specs=[
            pl.BlockSpec((1, rows, d), lambda i: (0, i, 0)),
            pl.BlockSpec((1, rows, d), lambda i: (1, i, 0)),
            pl.BlockSpec((rows, d), lambda i: (i, 0)),
            pl.BlockSpec((rows, 1), lambda i: (i, 0)),
            pl.BlockSpec((1, d), lambda i: (0, 0)),
            pl.BlockSpec((d, d_out), lambda i: (0, 0)),
        ],
        out_specs=pl.BlockSpec((rows, d_out), lambda i: (i, 0)),
        out_shape=jax.ShapeDtypeStruct((n, d_out), jnp.float32),
    )(acc, acc, xwp, dinv, b_in.reshape(1, d), w)


def _tc_final(acc, xwp, dinv, b_in, w, b_out, rows):
    n, d = xwp.shape
    d_out = w.shape[1]

    def body(a0_b, a1_b, xwp_b, dinv_b, b_b, w_b, bo_b, out_b):
        h = jnp.maximum(
            (a0_b[0] + a1_b[0] + xwp_b[...]) * dinv_b[...] + b_b[...], 0.0)
        z = jnp.dot(h, w_b[...], preferred_element_type=jnp.float32) + bo_b[...]
        out_b[...] = jax.nn.sigmoid(z)

    return pl.pallas_call(
        body,
        grid=(n // rows,),
        in_specs=[
            pl.BlockSpec((1, rows, d), lambda i: (0, i, 0)),
            pl.BlockSpec((1, rows, d), lambda i: (1, i, 0)),
            pl.BlockSpec((rows, d), lambda i: (i, 0)),
            pl.BlockSpec((rows, 1), lambda i: (i, 0)),
            pl.BlockSpec((1, d), lambda i: (0, 0)),
            pl.BlockSpec((d, d_out), lambda i: (0, 0)),
            pl.BlockSpec((1, d_out), lambda i: (0, 0)),
        ],
        out_specs=pl.BlockSpec((rows, d_out), lambda i: (i, 0)),
        out_shape=jax.ShapeDtypeStruct((n, d_out), jnp.float32),
    )(acc, acc, xwp, dinv, b_in.reshape(1, d), w, b_out.reshape(1, d_out))


def kernel(x, edge_index, W1, b1, W2, b2, Wlin, blin):
    n, _ = x.shape
    e = edge_index.shape[1]

    per_tile = e // _NW
    assert e % _NW == 0
    k_chunk = 125
    assert per_tile % k_chunk == 0
    n_chunks = per_tile // k_chunk
    n_blocks = 5 if (n_chunks // 5) % 8 == 0 else 1
    n_pad_deg = -(-n // (_NS * 128)) * (_NS * 128)
    n_pad_seg = -(-n // (_NS * 8)) * (_NS * 8)

    src_r = edge_index[0].reshape(_NW, n_chunks, k_chunk)
    dst_r = edge_index[1].reshape(_NW, n_chunks, k_chunk)

    rows = 1000
    assert n % rows == 0

    k_deg = 80
    dst_deg = edge_index[1].reshape(_NW, per_tile // k_deg, k_deg)
    deg_parts = _sc_degree(dst_deg, n_pad_deg, k_deg, per_tile // k_deg)
    deg_t = deg_parts.T[:n]

    xwp1, dinv = _tc_prescale(x, W1, deg_t, rows)
    acc1 = _sc_segment_sum(xwp1, src_r, dst_r, n_pad_seg, k_chunk,
                           n_chunks, n_blocks)
    xwp2 = _tc_mid(acc1, xwp1, dinv, b1, W2, rows)
    acc2 = _sc_segment_sum(xwp2, src_r, dst_r, n_pad_seg, k_chunk,
                           n_chunks, n_blocks)
    return _tc_final(acc2, xwp2, dinv, b2, Wlin, blin, rows)


# speedup vs baseline: 34.0757x; 1.2185x over previous
import functools

import jax
import jax.numpy as jnp
from jax import lax
from jax.experimental import pallas as pl
from jax.experimental.pallas import tpu as pltpu
from jax.experimental.pallas import tpu_sc as plsc

_NC = 2
_NS = 16
_NW = _NC * _NS
_L = 16


def _sc_mesh():
    return plsc.VectorSubcoreMesh(core_axis_name="c", subcore_axis_name="s")


def _fill(ref, n, value16):
    def body(i, _):
        ref[pl.ds(i * _L, _L)] = value16
        return 0
    lax.fori_loop(0, n // _L, body, 0)


def _sc_degree(dst_r, n_pad, k_chunk, n_chunks):
    rows_per_tile = n_pad // _NS
    zb_n = -(-rows_per_tile // _L) * _L

    @functools.partial(
        pl.kernel,
        out_type=jax.ShapeDtypeStruct((_NC, n_pad), jnp.float32),
        mesh=_sc_mesh(),
        scratch_types=[
            pltpu.VMEM((n_chunks, k_chunk), jnp.int32),
            pltpu.VMEM((k_chunk,), jnp.float32),
            pltpu.VMEM((zb_n,), jnp.float32),
            pltpu.VMEM_SHARED((n_pad,), jnp.float32),
        ],
    )
    def k(dst_hbm, out_hbm, dst_v, ones_v, zb, acc_sp):
        c = lax.axis_index("c")
        s = lax.axis_index("s")
        wid = c * _NS + s
        base = s * rows_per_tile
        _fill(ones_v, k_chunk, jnp.ones((_L,), jnp.float32))
        _fill(zb, zb_n, jnp.zeros((_L,), jnp.float32))
        pltpu.sync_copy(zb.at[pl.ds(0, rows_per_tile)],
                        acc_sp.at[pl.ds(base, rows_per_tile)])
        plsc.subcore_barrier()
        pltpu.sync_copy(dst_hbm.at[wid], dst_v)

        def body(j, _):
            pltpu.sync_copy(ones_v, acc_sp.at[dst_v.at[j]], add=True)
            return 0
        lax.fori_loop(0, n_chunks, body, 0)

        plsc.subcore_barrier()
        pltpu.sync_copy(acc_sp.at[pl.ds(base, rows_per_tile)],
                        out_hbm.at[c, pl.ds(base, rows_per_tile)])

    return k(dst_r)


def _sc_segment_sum(xwp, src_r, dst_r, n_pad, k_chunk, n_chunks, n_blocks,
                    n_buf):
    d = xwp.shape[1]
    rows_per_tile = n_pad // _NS
    sb = n_chunks // n_blocks
    assert sb % n_buf == 0

    @functools.partial(
        pl.kernel,
        out_type=jax.ShapeDtypeStruct((_NC, n_pad, d), jnp.float32),
        mesh=_sc_mesh(),
        scratch_types=[
            [pltpu.VMEM((sb, k_chunk), jnp.int32) for _ in range(2)],
            [pltpu.VMEM((sb, k_chunk), jnp.int32) for _ in range(2)],
            [pltpu.VMEM((k_chunk, d), jnp.float32) for _ in range(n_buf)],
            pltpu.VMEM_SHARED((n_pad, d), jnp.float32),
            [pltpu.SemaphoreType.DMA for _ in range(2 * 2 + n_buf)],
        ],
    )
    def k(xw_hbm, src_hbm, dst_hbm, out_hbm,
          srcbs, dstbs, rows, acc_sp, sems):
        c = lax.axis_index("c")
        s = lax.axis_index("s")
        wid = c * _NS + s
        base = s * rows_per_tile
        ssems, dsems, gsems = sems[0:2], sems[2:4], sems[4:]

        zero16 = jnp.zeros((_L,), jnp.float32)

        def zrow(i, _):
            def zcol(j, _):
                rows[0][i, pl.ds(j * _L, _L)] = zero16
                return 0
            lax.fori_loop(0, d // _L, zcol, 0)
            return 0
        lax.fori_loop(0, k_chunk, zrow, 0)
        zc = (k_chunk // 8) * 8
        done = 0
        while done < rows_per_tile:
            nrow = min(zc, rows_per_tile - done)
            pltpu.sync_copy(rows[0].at[pl.ds(0, nrow)],
                            acc_sp.at[pl.ds(base + done, nrow)])
            done += nrow
        plsc.subcore_barrier()

        def idx_block(hbm, b, buf, sem):
            return pltpu.make_async_copy(
                hbm.at[wid, pl.ds(b * sb, sb)], buf, sem)

        def gather(srcb, jj, v):
            return pltpu.make_async_copy(
                xw_hbm.at[srcb.at[jj]], rows[v], gsems[v])

        idx_block(src_hbm, 0, srcbs[0], ssems[0]).start()
        idx_block(dst_hbm, 0, dstbs[0], dsems[0]).start()
        for b in range(n_blocks):
            srcb, dstb = srcbs[b % 2], dstbs[b % 2]
            idx_block(src_hbm, b, srcb, ssems[b % 2]).wait()
            idx_block(dst_hbm, b, dstb, dsems[b % 2]).wait()
            if b + 1 < n_blocks:
                idx_block(src_hbm, b + 1, srcbs[(b + 1) % 2],
                          ssems[(b + 1) % 2]).start()
                idx_block(dst_hbm, b + 1, dstbs[(b + 1) % 2],
                          dsems[(b + 1) % 2]).start()

            for v in range(n_buf - 1):
                gather(srcb, v, v).start()

            def outer(i, _):
                j0 = i * n_buf
                for v in range(n_buf):
                    j = j0 + v
                    gather(srcb, j, v).wait()
                    nxt = j + n_buf - 1

                    @pl.when(nxt < sb)
                    def _():
                        gather(srcb, nxt, (v - 1) % n_buf).start()
                    pltpu.sync_copy(rows[v], acc_sp.at[dstb.at[j]], add=True)
                return 0
            lax.fori_loop(0, sb // n_buf, outer, 0)

        plsc.subcore_barrier()
        pltpu.sync_copy(acc_sp.at[pl.ds(base, rows_per_tile)],
                        out_hbm.at[c, pl.ds(base, rows_per_tile)])

    return k(xwp, src_r, dst_r)


def _tc_prescale(x, w1, deg_t, rows):
    n, d_in = x.shape
    d_out = w1.shape[1]

    def body(x_b, w_b, deg_b, xwp_b, dinv_b):
        deg = deg_b[:, 0:1] + deg_b[:, 1:2] + 1.0
        dinv = lax.rsqrt(deg)
        xw = jnp.dot(x_b[...], w_b[...], preferred_element_type=jnp.float32)
        xwp_b[...] = xw * dinv
        dinv_b[...] = dinv

    return pl.pallas_call(
        body,
        grid=(n // rows,),
        in_specs=[
            pl.BlockSpec((rows, d_in), lambda i: (i, 0)),
            pl.BlockSpec((d_in, d_out), lambda i: (0, 0)),
            pl.BlockSpec((rows, _NC), lambda i: (i, 0)),
        ],
        out_specs=[
            pl.BlockSpec((rows, d_out), lambda i: (i, 0)),
            pl.BlockSpec((rows, 1), lambda i: (i, 0)),
        ],
        out_shape=[
            jax.ShapeDtypeStruct((n, d_out), jnp.float32),
            jax.ShapeDtypeStruct((n, 1), jnp.float32),
        ],
    )(x, w1, deg_t)


def _tc_mid(acc, xwp, dinv, b_in, w, rows):
    n, d = xwp.shape
    n_pad = acc.shape[1]
    d_out = w.shape[1]

    def body(a0_b, a1_b, xwp_b, dinv_b, b_b, w_b, out_b):
        h = jnp.maximum(
            (a0_b[0] + a1_b[0] + xwp_b[...]) * dinv_b[...] + b_b[...], 0.0)
        out_b[...] = jnp.dot(h, w_b[...],
                             preferred_element_type=jnp.float32) * dinv_b[...]

    return pl.pallas_call(
        body,
        grid=(n // rows,),
        in_specs=[
            pl.BlockSpec((1, rows, d), lambda i: (0, i, 0)),
            pl.BlockSpec((1, rows, d), lambda i: (1, i, 0)),
            pl.BlockSpec((rows, d), lambda i: (i, 0)),
            pl.BlockSpec((rows, 1), lambda i: (i, 0)),
            pl.BlockSpec((1, d), lambda i: (0, 0)),
            pl.BlockSpec((d, d_out), lambda i: (0, 0)),
        ],
        out_specs=pl.BlockSpec((rows, d_out), lambda i: (i, 0)),
        out_shape=jax.ShapeDtypeStruct((n, d_out), jnp.float32),
    )(acc, acc, xwp, dinv, b_in.reshape(1, d), w)


def _tc_final(acc, xwp, dinv, b_in, w, b_out, rows):
    n, d = xwp.shape
    d_out = w.shape[1]

    def body(a0_b, a1_b, xwp_b, dinv_b, b_b, w_b, bo_b, out_b):
        h = jnp.maximum(
            (a0_b[0] + a1_b[0] + xwp_b[...]) * dinv_b[...] + b_b[...], 0.0)
        z = jnp.dot(h, w_b[...], preferred_element_type=jnp.float32) + bo_b[...]
        out_b[...] = jax.nn.sigmoid(z)

    return pl.pallas_call(
        body,
        grid=(n // rows,),
        in_specs=[
            pl.BlockSpec((1, rows, d), lambda i: (0, i, 0)),
            pl.BlockSpec((1, rows, d), lambda i: (1, i, 0)),
            pl.BlockSpec((rows, d), lambda i: (i, 0)),
            pl.BlockSpec((rows, 1), lambda i: (i, 0)),
            pl.BlockSpec((1, d), lambda i: (0, 0)),
            pl.BlockSpec((d, d_out), lambda i: (0, 0)),
            pl.BlockSpec((1, d_out), lambda i: (0, 0)),
        ],
        out_specs=pl.BlockSpec((rows, d_out), lambda i: (i, 0)),
        out_shape=jax.ShapeDtypeStruct((n, d_out), jnp.float32),
    )(acc, acc, xwp, dinv, b_in.reshape(1, d), w, b_out.reshape(1, d_out))


def kernel(x, edge_index, W1, b1, W2, b2, Wlin, blin):
    n, _ = x.shape
    e = edge_index.shape[1]

    per_tile = e // _NW
    assert e % _NW == 0
    k_chunk = 50
    assert per_tile % k_chunk == 0
    n_chunks = per_tile // k_chunk
    n_blocks = 5
    n_buf = 4
    assert (n_chunks // n_blocks) % 8 == 0
    n_pad_deg = -(-n // (_NS * 128)) * (_NS * 128)
    n_pad_seg = -(-n // (_NS * 8)) * (_NS * 8)

    src_r = edge_index[0].reshape(_NW, n_chunks, k_chunk)
    dst_r = edge_index[1].reshape(_NW, n_chunks, k_chunk)

    rows = 1000
    assert n % rows == 0

    k_deg = 80
    dst_deg = edge_index[1].reshape(_NW, per_tile // k_deg, k_deg)
    deg_parts = _sc_degree(dst_deg, n_pad_deg, k_deg, per_tile // k_deg)
    deg_t = deg_parts.T[:n]

    xwp1, dinv = _tc_prescale(x, W1, deg_t, rows)
    acc1 = _sc_segment_sum(xwp1, src_r, dst_r, n_pad_seg, k_chunk,
                           n_chunks, n_blocks, n_buf)
    xwp2 = _tc_mid(acc1, xwp1, dinv, b1, W2, rows)
    acc2 = _sc_segment_sum(xwp2, src_r, dst_r, n_pad_seg, k_chunk,
                           n_chunks, n_blocks, n_buf)
    return _tc_final(acc2, xwp2, dinv, b2, Wlin, blin, rows)


# speedup vs baseline: 34.7353x; 1.0194x over previous
import functools

import jax
import jax.numpy as jnp
from jax import lax
from jax.experimental import pallas as pl
from jax.experimental.pallas import tpu as pltpu
from jax.experimental.pallas import tpu_sc as plsc

_NC = 2
_NS = 16
_NW = _NC * _NS
_L = 16


def _sc_mesh():
    return plsc.VectorSubcoreMesh(core_axis_name="c", subcore_axis_name="s")


def _fill(ref, n, value16):
    def body(i, _):
        ref[pl.ds(i * _L, _L)] = value16
        return 0
    lax.fori_loop(0, n // _L, body, 0)


def _sc_degree(e4, n_pad, k_chunk, n_chunks):
    rows_per_tile = n_pad // _NS
    ones_n = -(-k_chunk // _L) * _L

    @functools.partial(
        pl.kernel,
        out_type=jax.ShapeDtypeStruct((_NC, n_pad), jnp.float32),
        mesh=_sc_mesh(),
        scratch_types=[
            pltpu.VMEM((n_chunks, k_chunk), jnp.int32),
            pltpu.VMEM((ones_n,), jnp.float32),
            pltpu.VMEM((rows_per_tile,), jnp.float32),
            pltpu.VMEM_SHARED((n_pad,), jnp.float32),
        ],
    )
    def k(e_hbm, out_hbm, dst_v, ones_v, zb, acc_sp):
        c = lax.axis_index("c")
        s = lax.axis_index("s")
        wid = c * _NS + s
        base = s * rows_per_tile
        _fill(ones_v, ones_n, jnp.ones((_L,), jnp.float32))
        _fill(zb, rows_per_tile, jnp.zeros((_L,), jnp.float32))
        pltpu.sync_copy(zb, acc_sp.at[pl.ds(base, rows_per_tile)])
        plsc.subcore_barrier()
        pltpu.sync_copy(e_hbm.at[1, wid], dst_v)

        def body(j, _):
            pltpu.sync_copy(ones_v.at[pl.ds(0, k_chunk)],
                            acc_sp.at[dst_v.at[j]], add=True)
            return 0
        lax.fori_loop(0, n_chunks, body, 0)

        plsc.subcore_barrier()
        pltpu.sync_copy(acc_sp.at[pl.ds(base, rows_per_tile)],
                        out_hbm.at[c, pl.ds(base, rows_per_tile)])

    return k(e4)


def _sc_segment_sum(xwp, e4, n_pad, k_chunk, n_chunks, n_blocks, n_buf):
    d = xwp.shape[1]
    rows_per_tile = n_pad // _NS
    sb = n_chunks // n_blocks
    assert sb % n_buf == 0

    @functools.partial(
        pl.kernel,
        out_type=jax.ShapeDtypeStruct((_NC, n_pad, d), jnp.float32),
        mesh=_sc_mesh(),
        scratch_types=[
            [pltpu.VMEM((sb, k_chunk), jnp.int32) for _ in range(2)],
            [pltpu.VMEM((sb, k_chunk), jnp.int32) for _ in range(2)],
            [pltpu.VMEM((k_chunk, d), jnp.float32) for _ in range(n_buf)],
            pltpu.VMEM_SHARED((n_pad, d), jnp.float32),
            [pltpu.SemaphoreType.DMA for _ in range(2 * 2 + n_buf)],
        ],
    )
    def k(xw_hbm, e_hbm, out_hbm,
          srcbs, dstbs, rows, acc_sp, sems):
        c = lax.axis_index("c")
        s = lax.axis_index("s")
        wid = c * _NS + s
        base = s * rows_per_tile
        ssems, dsems, gsems = sems[0:2], sems[2:4], sems[4:]

        zero16 = jnp.zeros((_L,), jnp.float32)

        def zrow(i, _):
            def zcol(j, _):
                rows[0][i, pl.ds(j * _L, _L)] = zero16
                return 0
            lax.fori_loop(0, d // _L, zcol, 0)
            return 0
        lax.fori_loop(0, k_chunk, zrow, 0)
        zc = (k_chunk // 8) * 8
        done = 0
        while done < rows_per_tile:
            nrow = min(zc, rows_per_tile - done)
            pltpu.sync_copy(rows[0].at[pl.ds(0, nrow)],
                            acc_sp.at[pl.ds(base + done, nrow)])
            done += nrow
        plsc.subcore_barrier()

        def idx_block(sd, b, buf, sem):
            return pltpu.make_async_copy(
                e_hbm.at[sd, wid, pl.ds(b * sb, sb)], buf, sem)

        def gather(srcb, jj, v):
            return pltpu.make_async_copy(
                xw_hbm.at[srcb.at[jj]], rows[v], gsems[v])

        idx_block(0, 0, srcbs[0], ssems[0]).start()
        idx_block(1, 0, dstbs[0], dsems[0]).start()
        for b in range(n_blocks):
            srcb, dstb = srcbs[b % 2], dstbs[b % 2]
            idx_block(0, b, srcb, ssems[b % 2]).wait()
            idx_block(1, b, dstb, dsems[b % 2]).wait()
            if b + 1 < n_blocks:
                idx_block(0, b + 1, srcbs[(b + 1) % 2],
                          ssems[(b + 1) % 2]).start()
                idx_block(1, b + 1, dstbs[(b + 1) % 2],
                          dsems[(b + 1) % 2]).start()

            for v in range(n_buf - 1):
                gather(srcb, v, v).start()

            def outer(i, _):
                j0 = i * n_buf
                for v in range(n_buf):
                    j = j0 + v
                    gather(srcb, j, v).wait()
                    nxt = j + n_buf - 1

                    @pl.when(nxt < sb)
                    def _():
                        gather(srcb, nxt, (v - 1) % n_buf).start()
                    pltpu.sync_copy(rows[v], acc_sp.at[dstb.at[j]], add=True)
                return 0
            lax.fori_loop(0, sb // n_buf, outer, 0)

        plsc.subcore_barrier()
        pltpu.sync_copy(acc_sp.at[pl.ds(base, rows_per_tile)],
                        out_hbm.at[c, pl.ds(base, rows_per_tile)])

    return k(xwp, e4)


def _tc_prescale(x, w1, deg_t, rows):
    n, d_in = x.shape
    d_out = w1.shape[1]

    def body(x_b, w_b, deg_b, xwp_b, dinv_b):
        deg = deg_b[:, 0:1] + deg_b[:, 1:2] + 1.0
        dinv = lax.rsqrt(deg)
        xw = jnp.dot(x_b[...], w_b[...], preferred_element_type=jnp.float32)
        xwp_b[...] = xw * dinv
        dinv_b[...] = dinv

    return pl.pallas_call(
        body,
        grid=(n // rows,),
        in_specs=[
            pl.BlockSpec((rows, d_in), lambda i: (i, 0)),
            pl.BlockSpec((d_in, d_out), lambda i: (0, 0)),
            pl.BlockSpec((rows, _NC), lambda i: (i, 0)),
        ],
        out_specs=[
            pl.BlockSpec((rows, d_out), lambda i: (i, 0)),
            pl.BlockSpec((rows, 1), lambda i: (i, 0)),
        ],
        out_shape=[
            jax.ShapeDtypeStruct((n, d_out), jnp.float32),
            jax.ShapeDtypeStruct((n, 1), jnp.float32),
        ],
    )(x, w1, deg_t)


def _tc_mid(acc, xwp, dinv, b_in, w, rows):
    n, d = xwp.shape
    n_pad = acc.shape[1]
    d_out = w.shape[1]

    def body(a0_b, a1_b, xwp_b, dinv_b, b_b, w_b, out_b):
        h = jnp.maximum(
            (a0_b[0] + a1_b[0] + xwp_b[...]) * dinv_b[...] + b_b[...], 0.0)
        out_b[...] = jnp.dot(h, w_b[...],
                             preferred_element_type=jnp.float32) * dinv_b[...]

    return pl.pallas_call(
        body,
        grid=(n // rows,),
        in_specs=[
            pl.BlockSpec((1, rows, d), lambda i: (0, i, 0)),
            pl.BlockSpec((1, rows, d), lambda i: (1, i, 0)),
            pl.BlockSpec((rows, d), lambda i: (i, 0)),
            pl.BlockSpec((rows, 1), lambda i: (i, 0)),
            pl.BlockSpec((1, d), lambda i: (0, 0)),
            pl.BlockSpec((d, d_out), lambda i: (0, 0)),
        ],
        out_specs=pl.BlockSpec((rows, d_out), lambda i: (i, 0)),
        out_shape=jax.ShapeDtypeStruct((n, d_out), jnp.float32),
    )(acc, acc, xwp, dinv, b_in.reshape(1, d), w)


def _tc_final(acc, xwp, dinv, b_in, w, b_out, rows):
    n, d = xwp.shape
    d_out = w.shape[1]

    def body(a0_b, a1_b, xwp_b, dinv_b, b_b, w_b, bo_b, out_b):
        h = jnp.maximum(
            (a0_b[0] + a1_b[0] + xwp_b[...]) * dinv_b[...] + b_b[...], 0.0)
        z = jnp.dot(h, w_b[...], preferred_element_type=jnp.float32) + bo_b[...]
        out_b[...] = jax.nn.sigmoid(z)

    return pl.pallas_call(
        body,
        grid=(n // rows,),
        in_specs=[
            pl.BlockSpec((1, rows, d), lambda i: (0, i, 0)),
            pl.BlockSpec((1, rows, d), lambda i: (1, i, 0)),
            pl.BlockSpec((rows, d), lambda i: (i, 0)),
            pl.BlockSpec((rows, 1), lambda i: (i, 0)),
            pl.BlockSpec((1, d), lambda i: (0, 0)),
            pl.BlockSpec((d, d_out), lambda i: (0, 0)),
            pl.BlockSpec((1, d_out), lambda i: (0, 0)),
        ],
        out_specs=pl.BlockSpec((rows, d_out), lambda i: (i, 0)),
        out_shape=jax.ShapeDtypeStruct((n, d_out), jnp.float32),
    )(acc, acc, xwp, dinv, b_in.reshape(1, d), w, b_out.reshape(1, d_out))


def kernel(x, edge_index, W1, b1, W2, b2, Wlin, blin):
    n, _ = x.shape
    e = edge_index.shape[1]

    per_tile = e // _NW
    assert e % _NW == 0
    k_chunk = 50
    assert per_tile % k_chunk == 0
    n_chunks = per_tile // k_chunk
    n_blocks = 5
    n_buf = 4
    assert (n_chunks // n_blocks) % 8 == 0
    n_pad_deg = -(-n // (_NS * 128)) * (_NS * 128)
    n_pad_seg = -(-n // (_NS * 8)) * (_NS * 8)

    e4 = edge_index.reshape(2, _NW, n_chunks, k_chunk)

    rows = 2000
    assert n % rows == 0

    deg_parts = _sc_degree(e4, n_pad_deg, k_chunk, n_chunks)
    deg_t = deg_parts.T[:n]

    xwp1, dinv = _tc_prescale(x, W1, deg_t, rows)
    acc1 = _sc_segment_sum(xwp1, e4, n_pad_seg, k_chunk,
                           n_chunks, n_blocks, n_buf)
    xwp2 = _tc_mid(acc1, xwp1, dinv, b1, W2, rows)
    acc2 = _sc_segment_sum(xwp2, e4, n_pad_seg, k_chunk,
                           n_chunks, n_blocks, n_buf)
    return _tc_final(acc2, xwp2, dinv, b2, Wlin, blin, rows)
